# baseline (speedup 1.0000x reference)
import jax
import jax.numpy as jnp
from jax import lax
from jax.experimental import pallas as pl
from jax.experimental.pallas import tpu as pltpu
from jax.experimental.pallas import tpu_sc as plsc

R0C = 5.0
NNODES = 10000
NPAD = 10112
NEDGES = 160000
DA = 128

BATCH = 80
SLICES = (32, 40, 56)
EPAD = 16 * BATCH * sum(SLICES)
ROWS_PER_TILE = NPAD // 16

BE = 2048


def _leaky(x):
    return jnp.maximum(x, 0.1 * x)


def _edge_body(rt, w0t, b0, w1t, b1, w2t, b2, w3t,
               p0, p1, p2, p3):
    x = rt[0:1, :]
    y = rt[1:2, :]
    z = rt[2:3, :]
    n2 = x * x + y * y + z * z
    xr = jnp.sqrt(n2 + 1e-12) * (1.0 / R0C)
    centers = lax.broadcasted_iota(jnp.int32, (8, 1), 0).astype(jnp.float32) * (1.0 / 7.0)
    d = xr - centers
    enc = jnp.transpose(jnp.exp(-32.0 * d * d), (1, 0))
    h = jnp.dot(enc, w0t[...], preferred_element_type=jnp.float32) + b0[...]
    h = _leaky(jnp.dot(h, w1t[...], preferred_element_type=jnp.float32) + b1[...])
    h = _leaky(jnp.dot(h, w2t[...], preferred_element_type=jnp.float32) + b2[...])
    rad = jnp.dot(h, w3t[...], preferred_element_type=jnp.float32)
    s = 7.0 / R0C
    inv = lax.rsqrt(1.0 + n2 * (s * s))
    srow = jnp.concatenate(
        [x * (s * inv), y * (s * inv), z * (s * inv),
         jnp.zeros((5, x.shape[1]), jnp.float32)], axis=0)
    scol = jnp.transpose(srow, (1, 0))
    p0[...] = rad
    p1[...] = rad * scol[:, 0:1]
    p2[...] = rad * scol[:, 1:2]
    p3[...] = rad * scol[:, 2:3]


def _edge_mlp(rt, w0t, b0, w1t, b1, w2t, b2, w3t):
    grid = rt.shape[1] // BE
    rspec = pl.BlockSpec((3, BE), lambda i: (0, i))
    full = lambda a: pl.BlockSpec(a.shape, lambda i: (0,) * a.ndim)
    out = pl.BlockSpec((BE, DA), lambda i: (i, 0))
    return pl.pallas_call(
        _edge_body,
        grid=(grid,),
        in_specs=[rspec,
                  full(w0t), full(b0), full(w1t), full(b1),
                  full(w2t), full(b2), full(w3t)],
        out_specs=[out, out, out, out],
        out_shape=[jax.ShapeDtypeStruct((rt.shape[1], DA), jnp.float32)] * 4,
    )(rt, w0t, b0, w1t, b1, w2t, b2, w3t)


NRING = 4


def _make_sc_body(nbatch):
  def _sc_body(p0, p1, p2, p3, i0, i1, i2, i3, src3d,
               o0, o1, o2, o3, buf0, buf1, buf2, buf3, idx, acc,
               gs0, gs1, gs2, gs3, ss0, ss1, ss2, ss3):
    bufs = (buf0, buf1, buf2, buf3)
    gsems = (gs0, gs1, gs2, gs3)
    ssems = (ss0, ss1, ss2, ss3)
    c = lax.axis_index("c")
    s = lax.axis_index("s")
    pltpu.sync_copy(src3d.at[s], idx)
    nds = pl.ds(s * ROWS_PER_TILE, ROWS_PER_TILE)
    ngrp = nbatch // NRING

    def do_chunk(p_hbm, init_hbm, out_hbm):
        def batch_ds(b):
            return pl.ds((s * nbatch + b) * BATCH, BATCH)

        def g_start(b, j):
            pltpu.async_copy(p_hbm.at[batch_ds(b)], bufs[j], gsems[j])

        def g_wait(j):
            pltpu.make_async_copy(p_hbm.at[batch_ds(0)], bufs[j], gsems[j]).wait()

        def s_start(b, j):
            pltpu.async_copy(bufs[j], acc.at[idx.at[b]], ssems[j], add=True)

        def s_wait(j):
            pltpu.make_async_copy(bufs[j], acc.at[idx.at[0]], ssems[j]).wait()

        for j in range(NRING):
            g_start(j, j)
        pltpu.sync_copy(init_hbm.at[nds], acc.at[nds])
        plsc.subcore_barrier()

        def body(g, carry):
            base = NRING * g
            for j in range(NRING):
                g_wait(j)
                s_start(base + j, j)

            @pl.when(g < ngrp - 1)
            def _():
                for j in range(NRING):
                    s_wait(j)
                    g_start(base + NRING + j, j)

            return carry

        lax.fori_loop(0, ngrp, body, 0)
        for j in range(NRING):
            s_wait(j)
        plsc.subcore_barrier()
        pltpu.sync_copy(acc.at[nds], out_hbm.at[nds])
        plsc.subcore_barrier()

    @pl.when(c == 0)
    def _():
        do_chunk(p0, i0, o0)
        do_chunk(p1, i1, o1)

    @pl.when(c == 1)
    def _():
        do_chunk(p2, i2, o2)
        do_chunk(p3, i3, o3)

  return _sc_body


def _sc_scatter(nbatch, p0, p1, p2, p3, i0, i1, i2, i3, src3d):
    mesh = plsc.VectorSubcoreMesh(core_axis_name="c", subcore_axis_name="s")
    fn = pl.kernel(
        _make_sc_body(nbatch),
        out_type=[jax.ShapeDtypeStruct((NPAD, DA), jnp.float32)] * 4,
        mesh=mesh,
        scratch_types=(
            [pltpu.VMEM((BATCH, DA), jnp.float32)] * NRING
            + [pltpu.VMEM((nbatch, BATCH), jnp.int32),
               pltpu.VMEM_SHARED((NPAD, DA), jnp.float32)]
            + [pltpu.SemaphoreType.DMA] * (2 * NRING)
        ),
    )
    return fn(p0, p1, p2, p3, i0, i1, i2, i3, src3d)


def _readout_body(a1, a2, a3, wvt, y0, y1, y2):
    y0[...] = jnp.dot(a1[...], wvt[...], preferred_element_type=jnp.float32)
    y1[...] = jnp.dot(a2[...], wvt[...], preferred_element_type=jnp.float32)
    y2[...] = jnp.dot(a3[...], wvt[...], preferred_element_type=jnp.float32)


def _readout(a1, a2, a3, wvt):
    bn = 632
    node = pl.BlockSpec((bn, DA), lambda i: (i, 0))
    wfull = pl.BlockSpec((DA, DA), lambda i: (0, 0))
    return pl.pallas_call(
        _readout_body,
        grid=(NPAD // bn,),
        in_specs=[node, node, node, wfull],
        out_specs=[node, node, node],
        out_shape=[jax.ShapeDtypeStruct((NPAD, DA), jnp.float32)] * 3,
    )(a1, a2, a3, wvt)


def kernel(graph, r_ij, W0, b0, W1, b1, W2, b2, W3, Wv):
    rt = jnp.pad(r_ij.T, ((0, 0), (0, EPAD - NEDGES)))
    srcp = jnp.pad(graph[0], (0, EPAD - NEDGES), constant_values=NNODES)
    wargs = (W0.T, b0.reshape(1, DA), W1.T, b1.reshape(1, DA),
             W2.T, b2.reshape(1, DA), W3.T)
    zeros = jnp.zeros((NPAD, DA), jnp.float32)
    accs = (zeros, zeros, zeros, zeros)
    e0 = 0
    for nb in SLICES:
        ne = 16 * BATCH * nb
        p = _edge_mlp(rt[:, e0:e0 + ne], *wargs)
        s3d = srcp[e0:e0 + ne].reshape(16, nb, BATCH)
        accs = _sc_scatter(nb, *p, *accs, s3d)
        e0 += ne
    a0, a1, a2, a3 = accs
    y0, y1, y2 = _readout(a1, a2, a3, Wv.T)
    out_v = jnp.stack([y0, y1, y2], axis=-1)[:NNODES]
    return a0[:NNODES], out_v


# speedup vs baseline: 1.0310x; 1.0310x over previous
import jax
import jax.numpy as jnp
from jax import lax
from jax.experimental import pallas as pl
from jax.experimental.pallas import tpu as pltpu
from jax.experimental.pallas import tpu_sc as plsc

R0C = 5.0
NNODES = 10000
NPAD = 10112
NEDGES = 160000
DA = 128

BATCH = 80
SLICES = (56, 72)
EPAD = 16 * BATCH * sum(SLICES)
ROWS_PER_TILE = NPAD // 16

BE = 2048


def _leaky(x):
    return jnp.maximum(x, 0.1 * x)


def _edge_body(rt, w0t, b0, w1t, b1, w2t, b2, w3t,
               p0, p1, p2, p3):
    x = rt[0:1, :]
    y = rt[1:2, :]
    z = rt[2:3, :]
    n2 = x * x + y * y + z * z
    xr = jnp.sqrt(n2 + 1e-12) * (1.0 / R0C)
    centers = lax.broadcasted_iota(jnp.int32, (8, 1), 0).astype(jnp.float32) * (1.0 / 7.0)
    d = xr - centers
    enc = jnp.transpose(jnp.exp(-32.0 * d * d), (1, 0))
    h = jnp.dot(enc, w0t[...], preferred_element_type=jnp.float32) + b0[...]
    h = _leaky(jnp.dot(h, w1t[...], preferred_element_type=jnp.float32) + b1[...])
    h = _leaky(jnp.dot(h, w2t[...], preferred_element_type=jnp.float32) + b2[...])
    rad = jnp.dot(h, w3t[...], preferred_element_type=jnp.float32)
    s = 7.0 / R0C
    inv = lax.rsqrt(1.0 + n2 * (s * s))
    srow = jnp.concatenate(
        [x * (s * inv), y * (s * inv), z * (s * inv),
         jnp.zeros((5, x.shape[1]), jnp.float32)], axis=0)
    scol = jnp.transpose(srow, (1, 0))
    p0[...] = rad
    p1[...] = rad * scol[:, 0:1]
    p2[...] = rad * scol[:, 1:2]
    p3[...] = rad * scol[:, 2:3]


def _edge_mlp(rt, w0t, b0, w1t, b1, w2t, b2, w3t):
    grid = rt.shape[1] // BE
    rspec = pl.BlockSpec((3, BE), lambda i: (0, i))
    full = lambda a: pl.BlockSpec(a.shape, lambda i: (0,) * a.ndim)
    out = pl.BlockSpec((BE, DA), lambda i: (i, 0))
    return pl.pallas_call(
        _edge_body,
        grid=(grid,),
        in_specs=[rspec,
                  full(w0t), full(b0), full(w1t), full(b1),
                  full(w2t), full(b2), full(w3t)],
        out_specs=[out, out, out, out],
        out_shape=[jax.ShapeDtypeStruct((rt.shape[1], DA), jnp.float32)] * 4,
    )(rt, w0t, b0, w1t, b1, w2t, b2, w3t)


NRING = 4


def _make_sc_body(nbatch):
  def _sc_body(p0, p1, p2, p3, i0, i1, i2, i3, src3d,
               o0, o1, o2, o3, buf0, buf1, buf2, buf3, idx, acc,
               gs0, gs1, gs2, gs3, ss0, ss1, ss2, ss3):
    bufs = (buf0, buf1, buf2, buf3)
    gsems = (gs0, gs1, gs2, gs3)
    ssems = (ss0, ss1, ss2, ss3)
    c = lax.axis_index("c")
    s = lax.axis_index("s")
    pltpu.sync_copy(src3d.at[s], idx)
    nds = pl.ds(s * ROWS_PER_TILE, ROWS_PER_TILE)
    ngrp = nbatch // NRING

    def do_chunk(p_hbm, init_hbm, out_hbm):
        def batch_ds(b):
            return pl.ds((s * nbatch + b) * BATCH, BATCH)

        def g_start(b, j):
            pltpu.async_copy(p_hbm.at[batch_ds(b)], bufs[j], gsems[j])

        def g_wait(j):
            pltpu.make_async_copy(p_hbm.at[batch_ds(0)], bufs[j], gsems[j]).wait()

        def s_start(b, j):
            pltpu.async_copy(bufs[j], acc.at[idx.at[b]], ssems[j], add=True)

        def s_wait(j):
            pltpu.make_async_copy(bufs[j], acc.at[idx.at[0]], ssems[j]).wait()

        for j in range(NRING):
            g_start(j, j)
        pltpu.sync_copy(init_hbm.at[nds], acc.at[nds])
        plsc.subcore_barrier()

        def body(g, carry):
            base = NRING * g
            for j in range(NRING):
                g_wait(j)
                s_start(base + j, j)

            @pl.when(g < ngrp - 1)
            def _():
                for j in range(NRING):
                    s_wait(j)
                    g_start(base + NRING + j, j)

            return carry

        lax.fori_loop(0, ngrp, body, 0)
        for j in range(NRING):
            s_wait(j)
        plsc.subcore_barrier()
        pltpu.sync_copy(acc.at[nds], out_hbm.at[nds])
        plsc.subcore_barrier()

    @pl.when(c == 0)
    def _():
        do_chunk(p0, i0, o0)
        do_chunk(p1, i1, o1)

    @pl.when(c == 1)
    def _():
        do_chunk(p2, i2, o2)
        do_chunk(p3, i3, o3)

  return _sc_body


def _sc_scatter(nbatch, p0, p1, p2, p3, i0, i1, i2, i3, src3d):
    mesh = plsc.VectorSubcoreMesh(core_axis_name="c", subcore_axis_name="s")
    fn = pl.kernel(
        _make_sc_body(nbatch),
        out_type=[jax.ShapeDtypeStruct((NPAD, DA), jnp.float32)] * 4,
        mesh=mesh,
        scratch_types=(
            [pltpu.VMEM((BATCH, DA), jnp.float32)] * NRING
            + [pltpu.VMEM((nbatch, BATCH), jnp.int32),
               pltpu.VMEM_SHARED((NPAD, DA), jnp.float32)]
            + [pltpu.SemaphoreType.DMA] * (2 * NRING)
        ),
    )
    return fn(p0, p1, p2, p3, i0, i1, i2, i3, src3d)


def _readout_body(a1, a2, a3, wvt, y0, y1, y2):
    y0[...] = jnp.dot(a1[...], wvt[...], preferred_element_type=jnp.float32)
    y1[...] = jnp.dot(a2[...], wvt[...], preferred_element_type=jnp.float32)
    y2[...] = jnp.dot(a3[...], wvt[...], preferred_element_type=jnp.float32)


def _readout(a1, a2, a3, wvt):
    bn = 632
    node = pl.BlockSpec((bn, DA), lambda i: (i, 0))
    wfull = pl.BlockSpec((DA, DA), lambda i: (0, 0))
    return pl.pallas_call(
        _readout_body,
        grid=(NPAD // bn,),
        in_specs=[node, node, node, wfull],
        out_specs=[node, node, node],
        out_shape=[jax.ShapeDtypeStruct((NPAD, DA), jnp.float32)] * 3,
    )(a1, a2, a3, wvt)


def kernel(graph, r_ij, W0, b0, W1, b1, W2, b2, W3, Wv):
    rt = jnp.pad(r_ij.T, ((0, 0), (0, EPAD - NEDGES)))
    srcp = jnp.pad(graph[0], (0, EPAD - NEDGES), constant_values=NNODES)
    wargs = (W0.T, b0.reshape(1, DA), W1.T, b1.reshape(1, DA),
             W2.T, b2.reshape(1, DA), W3.T)
    zeros = jnp.zeros((NPAD, DA), jnp.float32)
    accs = (zeros, zeros, zeros, zeros)
    e0 = 0
    for nb in SLICES:
        ne = 16 * BATCH * nb
        p = _edge_mlp(rt[:, e0:e0 + ne], *wargs)
        s3d = srcp[e0:e0 + ne].reshape(16, nb, BATCH)
        accs = _sc_scatter(nb, *p, *accs, s3d)
        e0 += ne
    a0, a1, a2, a3 = accs
    y0, y1, y2 = _readout(a1, a2, a3, Wv.T)
    out_v = jnp.stack([y0, y1, y2], axis=-1)[:NNODES]
    return a0[:NNODES], out_v


# speedup vs baseline: 1.0370x; 1.0058x over previous
import jax
import jax.numpy as jnp
from jax import lax
from jax.experimental import pallas as pl
from jax.experimental.pallas import tpu as pltpu
from jax.experimental.pallas import tpu_sc as plsc

R0C = 5.0
NNODES = 10000
NPAD = 10112
NEDGES = 160000
DA = 128

BATCH = 80
SLICES = (56, 72)
EPAD = 16 * BATCH * sum(SLICES)
ROWS_PER_TILE = NPAD // 16

BE = 2048


def _leaky(x):
    return jnp.maximum(x, 0.1 * x)


def _edge_body(rt, w0t, b0, w1t, b1, w2t, b2, w3t,
               p0, p1, p2, p3):
    x = rt[0:1, :]
    y = rt[1:2, :]
    z = rt[2:3, :]
    n2 = x * x + y * y + z * z
    xr = jnp.sqrt(n2 + 1e-12) * (1.0 / R0C)
    centers = lax.broadcasted_iota(jnp.int32, (8, 1), 0).astype(jnp.float32) * (1.0 / 7.0)
    d = xr - centers
    enc = jnp.transpose(jnp.exp(-32.0 * d * d), (1, 0))
    h = jnp.dot(enc, w0t[...], preferred_element_type=jnp.float32) + b0[...]
    h = _leaky(jnp.dot(h, w1t[...], preferred_element_type=jnp.float32) + b1[...])
    h = _leaky(jnp.dot(h, w2t[...], preferred_element_type=jnp.float32) + b2[...])
    rad = jnp.dot(h, w3t[...], preferred_element_type=jnp.float32)
    s = 7.0 / R0C
    inv = lax.rsqrt(1.0 + n2 * (s * s))
    srow = jnp.concatenate(
        [x * (s * inv), y * (s * inv), z * (s * inv),
         jnp.zeros((5, x.shape[1]), jnp.float32)], axis=0)
    scol = jnp.transpose(srow, (1, 0))
    p0[...] = rad
    p1[...] = rad * scol[:, 0:1]
    p2[...] = rad * scol[:, 1:2]
    p3[...] = rad * scol[:, 2:3]


def _edge_mlp(rt, w0t, b0, w1t, b1, w2t, b2, w3t):
    grid = rt.shape[1] // BE
    rspec = pl.BlockSpec((3, BE), lambda i: (0, i))
    full = lambda a: pl.BlockSpec(a.shape, lambda i: (0,) * a.ndim)
    out = pl.BlockSpec((BE, DA), lambda i: (i, 0))
    return pl.pallas_call(
        _edge_body,
        grid=(grid,),
        in_specs=[rspec,
                  full(w0t), full(b0), full(w1t), full(b1),
                  full(w2t), full(b2), full(w3t)],
        out_specs=[out, out, out, out],
        out_shape=[jax.ShapeDtypeStruct((rt.shape[1], DA), jnp.float32)] * 4,
    )(rt, w0t, b0, w1t, b1, w2t, b2, w3t)


NRING = 4


def _make_sc_body(nbatch):
  def _sc_body(p0, p1, p2, p3, i0, i1, i2, i3, src3d,
               o0, o1, o2, o3, buf0, buf1, buf2, buf3, idx, acc,
               gs0, gs1, gs2, gs3, ss0, ss1, ss2, ss3):
    bufs = (buf0, buf1, buf2, buf3)
    gsems = (gs0, gs1, gs2, gs3)
    ssems = (ss0, ss1, ss2, ss3)
    c = lax.axis_index("c")
    s = lax.axis_index("s")
    pltpu.sync_copy(src3d.at[s], idx)
    nds = pl.ds(s * ROWS_PER_TILE, ROWS_PER_TILE)
    ngrp = nbatch // NRING

    def do_chunk(p_hbm, init_hbm, out_hbm):
        def batch_ds(b):
            return pl.ds((s * nbatch + b) * BATCH, BATCH)

        def g_start(b, j):
            pltpu.async_copy(p_hbm.at[batch_ds(b)], bufs[j], gsems[j])

        def g_wait(j):
            pltpu.make_async_copy(p_hbm.at[batch_ds(0)], bufs[j], gsems[j]).wait()

        def s_start(b, j):
            pltpu.async_copy(bufs[j], acc.at[idx.at[b]], ssems[j], add=True)

        def s_wait(j):
            pltpu.make_async_copy(bufs[j], acc.at[idx.at[0]], ssems[j]).wait()

        for j in range(NRING):
            g_start(j, j)
        pltpu.sync_copy(init_hbm.at[nds], acc.at[nds])
        plsc.subcore_barrier()

        def body(g, carry):
            base = NRING * g
            for j in range(NRING):
                g_wait(j)
                s_start(base + j, j)

            @pl.when(g < ngrp - 1)
            def _():
                for j in range(NRING):
                    s_wait(j)
                    g_start(base + NRING + j, j)

            return carry

        lax.fori_loop(0, ngrp, body, 0)
        for j in range(NRING):
            s_wait(j)
        plsc.subcore_barrier()
        pltpu.sync_copy(acc.at[nds], out_hbm.at[nds])
        plsc.subcore_barrier()

    @pl.when(c == 0)
    def _():
        do_chunk(p0, i0, o0)
        do_chunk(p1, i1, o1)

    @pl.when(c == 1)
    def _():
        do_chunk(p2, i2, o2)
        do_chunk(p3, i3, o3)

  return _sc_body


def _sc_scatter(nbatch, p0, p1, p2, p3, i0, i1, i2, i3, src3d):
    mesh = plsc.VectorSubcoreMesh(core_axis_name="c", subcore_axis_name="s")
    fn = pl.kernel(
        _make_sc_body(nbatch),
        out_type=[jax.ShapeDtypeStruct((NPAD, DA), jnp.float32)] * 4,
        mesh=mesh,
        scratch_types=(
            [pltpu.VMEM((BATCH, DA), jnp.float32)] * NRING
            + [pltpu.VMEM((nbatch, BATCH), jnp.int32),
               pltpu.VMEM_SHARED((NPAD, DA), jnp.float32)]
            + [pltpu.SemaphoreType.DMA] * (2 * NRING)
        ),
    )
    return fn(p0, p1, p2, p3, i0, i1, i2, i3, src3d)


def _readout_body(a0, a1, a2, a3, wvt, aa, y0, y1, y2):
    aa[...] = a0[...]
    y0[...] = jnp.dot(a1[...], wvt[...], preferred_element_type=jnp.float32)
    y1[...] = jnp.dot(a2[...], wvt[...], preferred_element_type=jnp.float32)
    y2[...] = jnp.dot(a3[...], wvt[...], preferred_element_type=jnp.float32)


def _readout(a0, a1, a2, a3, wvt):
    bn = 632
    node = pl.BlockSpec((bn, DA), lambda i: (i, 0))
    wfull = pl.BlockSpec((DA, DA), lambda i: (0, 0))
    return pl.pallas_call(
        _readout_body,
        grid=(NPAD // bn,),
        in_specs=[node, node, node, node, wfull],
        out_specs=[node, node, node, node],
        out_shape=[jax.ShapeDtypeStruct((NNODES, DA), jnp.float32)] * 4,
    )(a0, a1, a2, a3, wvt)


def kernel(graph, r_ij, W0, b0, W1, b1, W2, b2, W3, Wv):
    rt = jnp.pad(r_ij.T, ((0, 0), (0, EPAD - NEDGES)))
    srcp = jnp.pad(graph[0], (0, EPAD - NEDGES), constant_values=NNODES)
    wargs = (W0.T, b0.reshape(1, DA), W1.T, b1.reshape(1, DA),
             W2.T, b2.reshape(1, DA), W3.T)
    zeros = jnp.zeros((NPAD, DA), jnp.float32)
    accs = (zeros, zeros, zeros, zeros)
    e0 = 0
    for nb in SLICES:
        ne = 16 * BATCH * nb
        p = _edge_mlp(rt[:, e0:e0 + ne], *wargs)
        s3d = srcp[e0:e0 + ne].reshape(16, nb, BATCH)
        accs = _sc_scatter(nb, *p, *accs, s3d)
        e0 += ne
    aa, y0, y1, y2 = _readout(*accs, Wv.T)
    out_v = jnp.stack([y0, y1, y2], axis=-1)
    return aa, out_v
